# baseline (speedup 1.0000x reference)
import functools

import jax
import jax.numpy as jnp
from jax import lax
from jax.experimental import pallas as pl
from jax.experimental.pallas import tpu as pltpu
from jax.experimental.pallas import tpu_sc as plsc

_MASK_RATIO = 0.75
_LANES = 16
_FULL_STEPS = 6
_TOTAL_STEPS = 30


@functools.lru_cache(maxsize=None)
def _build_mask_kernel(B, N, K):
    NW = 32
    rows_per_w = B // NW
    n_chunks = N // _LANES
    sentinel = 1 << _TOTAL_STEPS
    mesh = plsc.VectorSubcoreMesh(core_axis_name="c", subcore_axis_name="s")

    cand_types = [pltpu.VMEM((N + _LANES,), jnp.int32)
                  for _ in range(rows_per_w)]

    @functools.partial(
        pl.kernel,
        mesh=mesh,
        out_type=jax.ShapeDtypeStruct((B, N), jnp.float32),
        compiler_params=pltpu.CompilerParams(needs_layout_passes=False),
        scratch_types=[
            pltpu.VMEM((rows_per_w, N), jnp.float32),
            pltpu.VMEM((rows_per_w, N), jnp.float32),
        ] + cand_types,
    )
    def body(noise_hbm, out_hbm, noise_v, out_v, *cand):
        wid = lax.axis_index("s") * 2 + lax.axis_index("c")
        base = wid * rows_per_w
        pltpu.sync_copy(noise_hbm.at[pl.ds(base, rows_per_w)], noise_v)

        def chunk(r, c):
            return plsc.bitcast(noise_v[r, pl.ds(c * _LANES, _LANES)],
                                jnp.int32)

        zero16 = jnp.zeros((_LANES,), jnp.int32)
        rows = range(rows_per_w)

        sent16 = jnp.full((_LANES,), sentinel, jnp.int32)
        scope = jax.named_scope
        for c in range(n_chunks + 1):
            for r in rows:
                cand[r][pl.ds(c * _LANES, _LANES)] = sent16

        def full_step(i, carry):
            lo, hi, cb = [list(t) for t in carry]
            mid = [lo[r] + ((hi[r] - lo[r]) >> 1) for r in rows]
            acc = [[zero16, zero16] for _ in rows]
            for c in range(n_chunks):
                for r in rows:
                    pc = plsc.all_reduce_population_count(chunk(r, c) <= mid[r])
                    acc[r][c & 1] = acc[r][c & 1] + pc
            for r in rows:
                cnt = acc[r][0] + acc[r][1]
                ge = cnt >= K
                lo[r] = jnp.where(ge, lo[r], mid[r] + 1)
                hi[r] = jnp.where(ge, mid[r], hi[r])
                cb[r] = jnp.where(ge, cb[r], cnt)
            return (tuple(lo), tuple(hi), tuple(cb))

        _s1 = scope("p1_bisect_full"); _s1.__enter__()
        init = (tuple(zero16 for _ in rows),
                tuple(jnp.full((_LANES,), sentinel - 1, jnp.int32)
                      for _ in rows),
                tuple(zero16 for _ in rows))
        lo, hi, cb = [list(t) for t in lax.fori_loop(
            0, _FULL_STEPS, full_step, init)]

        _s1.__exit__(None, None, None)
        _s2 = scope("p2_compact"); _s2.__enter__()
        c0 = [cb[r] for r in rows]
        off = [zero16 for _ in rows]
        for c in range(n_chunks):
            for r in rows:
                k = chunk(r, c)
                m = (k >= lo[r]) & (k <= hi[r])
                mi = m.astype(jnp.int32)
                idx = off[r] + jnp.cumsum(mi) - mi
                plsc.store_scatter(cand[r], [idx], k, mask=m)
                off[r] = off[r] + plsc.all_reduce_population_count(m)

        nc_v = off[0]
        for r in rows:
            if r:
                nc_v = jnp.maximum(nc_v, off[r])
        nc = (nc_v[0] + _LANES - 1) // _LANES

        _s2.__exit__(None, None, None)
        _s3 = scope("p3_bisect_window"); _s3.__enter__()
        def win_step(i, carry):
            lo, hi, cb = [list(t) for t in carry]
            mid = [lo[r] + ((hi[r] - lo[r]) >> 1) for r in rows]

            def wbody(j, accs):
                out = []
                for r in rows:
                    k = cand[r][pl.ds(j * _LANES, _LANES)]
                    out.append(accs[r]
                               + plsc.all_reduce_population_count(k <= mid[r]))
                return tuple(out)

            accs = lax.fori_loop(0, nc, wbody,
                                 tuple(zero16 for _ in rows))
            for r in rows:
                cnt = c0[r] + accs[r]
                ge = cnt >= K
                lo[r] = jnp.where(ge, lo[r], mid[r] + 1)
                hi[r] = jnp.where(ge, mid[r], hi[r])
                cb[r] = jnp.where(ge, cb[r], cnt)
            return (tuple(lo), tuple(hi), tuple(cb))

        lo, hi, cb = [list(t) for t in lax.fori_loop(
            0, _TOTAL_STEPS - _FULL_STEPS, win_step,
            (tuple(lo), tuple(hi), tuple(cb)))]

        vstar = lo
        rem = [K - cb[r] for r in rows]

        _s3.__exit__(None, None, None)
        _s4 = scope("p4_mask"); _s4.__enter__()
        carry = [zero16 for _ in rows]
        for c in range(n_chunks):
            for r in rows:
                k = chunk(r, c)
                eq = k == vstar[r]
                eqi = eq.astype(jnp.int32)
                excl = jnp.cumsum(eqi) - eqi + carry[r]
                vis = (k < vstar[r]) | (eq & (excl < rem[r]))
                out_v[r, pl.ds(c * _LANES, _LANES)] = vis.astype(jnp.float32)
                carry[r] = carry[r] + plsc.all_reduce_population_count(eq)

        _s4.__exit__(None, None, None)
        pltpu.sync_copy(out_v, out_hbm.at[pl.ds(base, rows_per_w)])

    return body


def kernel(x, noise):
    B, N = x.shape[0], x.shape[1]
    num_visible = int(round(N * (1.0 - _MASK_RATIO)))
    num_visible = min(max(1, num_visible), N - 1)
    return _build_mask_kernel(B, N, num_visible)(noise)


# speedup vs baseline: 1.0758x; 1.0758x over previous
import functools

import jax
import jax.numpy as jnp
from jax import lax
from jax.experimental import pallas as pl
from jax.experimental.pallas import tpu as pltpu
from jax.experimental.pallas import tpu_sc as plsc

_MASK_RATIO = 0.75
_LANES = 16
_FULL_STEPS = 8
_TOTAL_STEPS = 30


@functools.lru_cache(maxsize=None)
def _build_mask_kernel(B, N, K):
    NW = 32
    rows_per_w = B // NW
    n_chunks = N // _LANES
    sentinel = 1 << _TOTAL_STEPS
    mesh = plsc.VectorSubcoreMesh(core_axis_name="c", subcore_axis_name="s")

    cand_types = [pltpu.VMEM((N + _LANES,), jnp.int32)
                  for _ in range(rows_per_w)]

    @functools.partial(
        pl.kernel,
        mesh=mesh,
        out_type=jax.ShapeDtypeStruct((B, N), jnp.float32),
        compiler_params=pltpu.CompilerParams(needs_layout_passes=False),
        scratch_types=[
            pltpu.VMEM((rows_per_w, N), jnp.float32),
            pltpu.VMEM((rows_per_w, N), jnp.float32),
        ] + cand_types,
    )
    def body(noise_hbm, out_hbm, noise_v, out_v, *cand):
        wid = lax.axis_index("s") * 2 + lax.axis_index("c")
        base = wid * rows_per_w
        pltpu.sync_copy(noise_hbm.at[pl.ds(base, rows_per_w)], noise_v)

        def chunk(r, c):
            return plsc.bitcast(noise_v[r, pl.ds(c * _LANES, _LANES)],
                                jnp.int32)

        def pcnt(m):
            return plsc.all_reduce_population_count(m)

        zero16 = jnp.zeros((_LANES,), jnp.int32)
        rows = range(rows_per_w)

        sent16 = jnp.full((_LANES,), sentinel, jnp.int32)
        for c in range(n_chunks + 1):
            for r in rows:
                cand[r][pl.ds(c * _LANES, _LANES)] = sent16

        def full_step(i, carry):
            lo, hi, cb = [list(t) for t in carry]
            mid = [lo[r] + ((hi[r] - lo[r]) >> 1) for r in rows]
            acc = [[zero16, zero16] for _ in rows]
            for c in range(n_chunks):
                for r in rows:
                    acc[r][c & 1] = acc[r][c & 1] + pcnt(chunk(r, c) <= mid[r])
            for r in rows:
                cnt = acc[r][0] + acc[r][1]
                ge = cnt >= K
                lo[r] = jnp.where(ge, lo[r], mid[r] + 1)
                hi[r] = jnp.where(ge, mid[r], hi[r])
                cb[r] = jnp.where(ge, cb[r], cnt)
            return (tuple(lo), tuple(hi), tuple(cb))

        with jax.named_scope("p1_bisect_full"):
            init = (tuple(zero16 for _ in rows),
                    tuple(jnp.full((_LANES,), sentinel - 1, jnp.int32)
                          for _ in rows),
                    tuple(zero16 for _ in rows))
            lo, hi, cb = [list(t) for t in lax.fori_loop(
                0, _FULL_STEPS, full_step, init)]

        with jax.named_scope("p2_compact"):
            c0 = [cb[r] for r in rows]
            off = [jnp.int32(0) for _ in rows]
            for c in range(n_chunks):
                for r in rows:
                    k = chunk(r, c)
                    m = (k >= lo[r]) & (k <= hi[r])
                    plsc.store_compressed(cand[r].at[pl.ds(off[r], _LANES)],
                                          k, mask=m)
                    off[r] = off[r] + pcnt(m)[0]

        with jax.named_scope("p3_bisect_window"):
            n_win = _TOTAL_STEPS - _FULL_STEPS

            for r in rows:
                c0r, Mr = c0[r], off[r]

                def upd(carry, mid, cnt):
                    lo, hi, cb = carry
                    ge = cnt >= K
                    return (jnp.where(ge, lo, mid + 1),
                            jnp.where(ge, mid, hi),
                            jnp.where(ge, cb, cnt))

                def fast(lo, hi, cb):
                    kw = cand[r][pl.ds(0, _LANES)]

                    def step(i, carry):
                        mid = carry[0] + ((carry[1] - carry[0]) >> 1)
                        return upd(carry, mid, c0r + pcnt(kw <= mid))

                    return lax.fori_loop(0, n_win, step, (lo, hi, cb))

                def slow(lo, hi, cb):
                    nc = (Mr + _LANES - 1) // _LANES

                    def step(i, carry):
                        mid = carry[0] + ((carry[1] - carry[0]) >> 1)

                        def wbody(j, acc):
                            kw = cand[r][pl.ds(j * _LANES, _LANES)]
                            return acc + pcnt(kw <= mid)

                        cnt = c0r + lax.fori_loop(0, nc, wbody, zero16)
                        return upd(carry, mid, cnt)

                    return lax.fori_loop(0, n_win, step, (lo, hi, cb))

                lo[r], hi[r], cb[r] = lax.cond(
                    Mr <= _LANES, fast, slow, lo[r], hi[r], cb[r])

        vstar = lo
        rem = [K - cb[r] for r in rows]

        with jax.named_scope("p4_mask"):
            carry = [zero16 for _ in rows]
            for c in range(n_chunks):
                for r in rows:
                    k = chunk(r, c)
                    eq = k == vstar[r]
                    eqi = eq.astype(jnp.int32)
                    excl = jnp.cumsum(eqi) - eqi + carry[r]
                    vis = (k < vstar[r]) | (eq & (excl < rem[r]))
                    out_v[r, pl.ds(c * _LANES, _LANES)] = (
                        vis.astype(jnp.float32))
                    carry[r] = carry[r] + pcnt(eq)

        pltpu.sync_copy(out_v, out_hbm.at[pl.ds(base, rows_per_w)])

    return body


def kernel(x, noise):
    B, N = x.shape[0], x.shape[1]
    num_visible = int(round(N * (1.0 - _MASK_RATIO)))
    num_visible = min(max(1, num_visible), N - 1)
    return _build_mask_kernel(B, N, num_visible)(noise)
